# baseline (speedup 1.0000x reference)
import jax
import jax.numpy as jnp
from jax.experimental import pallas as pl
from jax.experimental.pallas import tpu as pltpu


_U = 8


def _feature_kernel(x_ref, w_ref, h_ref):
    h_ref[...] = jnp.dot(
        x_ref[...], w_ref[...], preferred_element_type=jnp.float32
    ).astype(h_ref.dtype)


def _make_deg_kernel(n_nodes, q, nbits, unroll):
    mask_n = (1 << nbits) - 1
    nb_full = q // unroll

    def _deg_kernel(packed_ref, deg_ref, da_ref, db_ref):
        z = pl.program_id(1)

        @pl.when(z == 0)
        def _():
            def _zero(i, c):
                da_ref[i] = 0
                db_ref[i] = 0
                return c
            jax.lax.fori_loop(0, n_nodes, _zero, 0)

        def _one(refs, v):
            d = v & mask_n
            w = 1 - (v >> 30)
            refs[d] = refs[d] + w

        def _body(j, c):
            base = j * unroll
            vs = [packed_ref[0, 0, base + u] for u in range(unroll)]
            for u in range(unroll):
                _one(da_ref if u % 2 == 0 else db_ref, vs[u])
            return c

        jax.lax.fori_loop(0, nb_full, _body, 0)

        def _tail(i, c):
            _one(da_ref, packed_ref[0, 0, i])
            return c

        jax.lax.fori_loop(nb_full * unroll, q, _tail, 0)

        @pl.when(z == pl.num_programs(1) - 1)
        def _():
            def _merge(i, c):
                deg_ref[0, 0, i] = da_ref[i] + db_ref[i]
                return c
            jax.lax.fori_loop(0, n_nodes, _merge, 0)

    return _deg_kernel


def _make_agg_kernel(q, nbits, p_h, p_o, chunk, n_e):
    mask_n = (1 << nbits) - 1

    def _agg_kernel(packed_ref, dinv_ref, h_ref, out_ref, acc_ref):
        z = pl.program_id(1)

        @pl.when(z == 0)
        def _():
            acc_ref[...] = jnp.zeros(acc_ref.shape, acc_ref.dtype)

        @pl.when(z < n_e)
        def _():
            def _decode(v):
                d = v & mask_n
                s = (v >> nbits) & mask_n
                w0 = v >> 30
                n = (dinv_ref[s] * dinv_ref[d]
                     * (1 - w0).astype(jnp.float32))
                slab = h_ref[pl.ds(pl.multiple_of(s * p_h, p_h), p_h), :]
                hrow = pltpu.bitcast(slab, jnp.bfloat16).astype(
                    jnp.float32) * n
                return d, hrow

            def _body(j, c):
                base = j * _U
                vs = [packed_ref[0, 0, base + u] for u in range(_U)]
                rows = [_decode(v) for v in vs]
                for d, hrow in rows:
                    o = pl.ds(pl.multiple_of(d * p_o, p_o), p_o)
                    acc_ref[o, :] = acc_ref[o, :] + hrow
                return c

            nb_full = q // _U
            jax.lax.fori_loop(0, nb_full, _body, 0)

            def _tail(i, c):
                d, hrow = _decode(packed_ref[0, 0, i])
                o = pl.ds(pl.multiple_of(d * p_o, p_o), p_o)
                acc_ref[o, :] = acc_ref[o, :] + hrow
                return c

            jax.lax.fori_loop(nb_full * _U, q, _tail, 0)

        @pl.when(z >= n_e)
        def _():
            zc = (z - n_e) * chunk
            out_ref[0, :, :] = acc_ref[pl.ds(zc, chunk), :]

    return _agg_kernel


def kernel(x, edge_index, weight, bias):
    N, nin = x.shape
    nout = weight.shape[1]
    E = edge_index.shape[1]
    nbits = (N - 1).bit_length()
    p_h = nout // 256
    p_o = nout // 128

    n_e = 2
    e_tot = E + N
    q = (e_tot + 2 * n_e - 1) // (2 * n_e)
    pad = 2 * n_e * q - e_tot

    src = edge_index[0].astype(jnp.int32)
    dst = edge_index[1].astype(jnp.int32)
    keep = src != dst
    loop = jnp.arange(N, dtype=jnp.int32)
    src_a = jnp.concatenate([src, loop])
    dst_a = jnp.concatenate([dst, loop])
    ew_off = jnp.concatenate(
        [jnp.where(keep, 0, 1 << 30).astype(jnp.int32),
         jnp.zeros((N,), jnp.int32)])
    packed = (src_a << nbits) | dst_a | ew_off
    if pad:
        packed = jnp.concatenate(
            [packed, jnp.full((pad,), 1 << 30, jnp.int32)])
    packed4 = packed.reshape(2 * n_e, 1, q)

    deg2 = pl.pallas_call(
        _make_deg_kernel(N, q, nbits, _U),
        out_shape=jax.ShapeDtypeStruct((2, 1, N), jnp.int32),
        grid_spec=pltpu.PrefetchScalarGridSpec(
            num_scalar_prefetch=0,
            grid=(2, n_e),
            in_specs=[
                pl.BlockSpec((1, 1, q), lambda g, z: (g * n_e + z, 0, 0),
                             memory_space=pltpu.SMEM),
            ],
            out_specs=pl.BlockSpec(
                (1, 1, N), lambda g, z: (g, 0, 0),
                memory_space=pltpu.SMEM),
            scratch_shapes=[
                pltpu.SMEM((N,), jnp.int32),
                pltpu.SMEM((N,), jnp.int32)],
        ),
        compiler_params=pltpu.CompilerParams(
            dimension_semantics=("parallel", "arbitrary")),
    )(packed4)

    deg = (deg2[0, 0] + deg2[1, 0]).astype(jnp.float32)
    dinv = jnp.where(deg > 0, jax.lax.rsqrt(deg), 0.0)

    xb = x.astype(jnp.bfloat16)
    wb = weight.astype(jnp.bfloat16)
    bm = min(N, 1024)
    hmat = pl.pallas_call(
        _feature_kernel,
        out_shape=jax.ShapeDtypeStruct((N, nout), jnp.bfloat16),
        grid=(N // bm,),
        in_specs=[
            pl.BlockSpec((bm, nin), lambda i: (i, 0)),
            pl.BlockSpec((nin, nout), lambda i: (0, 0)),
        ],
        out_specs=pl.BlockSpec((bm, nout), lambda i: (i, 0)),
        compiler_params=pltpu.CompilerParams(
            dimension_semantics=("parallel",)),
    )(xb, wb)

    h_i32 = jax.lax.bitcast_convert_type(
        hmat.reshape(N, p_h, 2, 128).transpose(0, 1, 3, 2), jnp.int32
    ).reshape(N * p_h, 128)

    n_chunks = 16
    chunk = (N * p_o) // n_chunks
    out2 = pl.pallas_call(
        _make_agg_kernel(q, nbits, p_h, p_o, chunk, n_e),
        out_shape=jax.ShapeDtypeStruct((2, N * p_o, 128), jnp.float32),
        grid_spec=pltpu.PrefetchScalarGridSpec(
            num_scalar_prefetch=0,
            grid=(2, n_e + n_chunks),
            in_specs=[
                pl.BlockSpec(
                    (1, 1, q),
                    lambda g, z: (g * n_e + jnp.minimum(z, n_e - 1), 0, 0),
                    memory_space=pltpu.SMEM),
                pl.BlockSpec(memory_space=pltpu.SMEM),
                pl.BlockSpec((N * p_h, 128), lambda g, z: (0, 0)),
            ],
            out_specs=pl.BlockSpec(
                (1, chunk, 128),
                lambda g, z: (g, jnp.maximum(z - n_e, 0), 0)),
            scratch_shapes=[
                pltpu.VMEM((N * p_o, 128), jnp.float32)],
        ),
        compiler_params=pltpu.CompilerParams(
            dimension_semantics=("parallel", "arbitrary"),
            vmem_limit_bytes=56 * 1024 * 1024),
    )(packed4, dinv, h_i32)

    out = (out2[0] + out2[1]).reshape(N, nout) + bias[None, :].astype(
        jnp.float32)
    return out
